# Initial kernel scaffold; baseline (speedup 1.0000x reference)
import jax
import jax.numpy as jnp
from jax.experimental import pallas as pl


def kernel(x, edge_index, batch, W_l1, b_l1, W_r1, W_l2, b_l2, W_r2, Wq1, bq1, Wk1, bk1, Wv1, bv1, Ws1, bs1, Wq2, bq2, Wk2, bk2, Wv2, bv2, Ws2, bs2, Wp1, bp1, Wp2, bp2):
    raise NotImplementedError("write your pallas kernel here")


# speedup vs baseline: 1.0002x; 1.0002x over previous
import functools
import math

import jax
import jax.numpy as jnp
from jax import lax
from jax.experimental import pallas as pl
from jax.experimental.pallas import tpu as pltpu

N = 10000
E = 320000
G = 16
H = 8
C = 32

_ROWS = 1000


def _mm_kernel(x_ref, w_ref, b_ref, o_ref, *, act):
    acc = jnp.dot(x_ref[...], w_ref[...], preferred_element_type=jnp.float32)
    acc = acc + b_ref[...]
    if act == "relu":
        acc = jnp.maximum(acc, 0.0)
    elif act == "sigmoid":
        acc = jax.nn.sigmoid(acc)
    o_ref[...] = acc


def _mm(x, w, b, act="none"):
    n, k = x.shape
    f = w.shape[1]
    grid = (n // _ROWS,)
    return pl.pallas_call(
        functools.partial(_mm_kernel, act=act),
        grid=grid,
        in_specs=[
            pl.BlockSpec((_ROWS, k), lambda i: (i, 0)),
            pl.BlockSpec((k, f), lambda i: (0, 0)),
            pl.BlockSpec((f,), lambda i: (0,)),
        ],
        out_specs=pl.BlockSpec((_ROWS, f), lambda i: (i, 0)),
        out_shape=jax.ShapeDtypeStruct((n, f), jnp.float32),
    )(x, w, b)


def _sage(x, src, dst, Wl, bl, Wr):
    ones = jnp.ones((src.shape[0],), x.dtype)
    deg = jax.ops.segment_sum(ones, dst, num_segments=N)
    agg = jax.ops.segment_sum(x[src], dst, num_segments=N)
    mean = agg / jnp.clip(deg, 1.0, None)[:, None]
    return jnp.maximum(_mm(mean, Wl, bl) + _mm(x, Wr, jnp.zeros_like(bl)), 0.0)


def _tconv(x, src, dst, Wq, bq, Wk, bk, Wv, bv, Ws, bs):
    q = _mm(x, Wq, bq).reshape(N, H, C)
    k = _mm(x, Wk, bk).reshape(N, H, C)
    v = _mm(x, Wv, bv).reshape(N, H, C)
    alpha = (q[dst] * k[src]).sum(-1) / jnp.sqrt(jnp.float32(C))
    amax = jax.ops.segment_max(alpha, dst, num_segments=N)
    amax = jnp.where(jnp.isfinite(amax), amax, 0.0)
    ex = jnp.exp(alpha - amax[dst])
    denom = jax.ops.segment_sum(ex, dst, num_segments=N)
    msg = jax.ops.segment_sum(ex[:, :, None] * v[src], dst, num_segments=N)
    out = msg / (denom[:, :, None] + 1e-16)
    return jnp.maximum(out.reshape(N, H * C) + _mm(x, Ws, bs), 0.0)


def _pool_kernel(h_ref, seg_ref, o_ref):
    i = pl.program_id(0)

    @pl.when(i == 0)
    def _init():
        o_ref[...] = jnp.full_like(o_ref, -jnp.inf)

    h = h_ref[...]
    seg = seg_ref[...]
    for g in range(G):
        vals = jnp.where(seg == g, h, -jnp.inf)
        mg = jnp.max(vals, axis=0, keepdims=True)
        o_ref[pl.ds(g, 1), :] = jnp.maximum(o_ref[pl.ds(g, 1), :], mg)


def _pool(h, batch):
    f = h.shape[1]
    seg = jnp.broadcast_to(batch.astype(jnp.int32)[:, None], (N, f))
    return pl.pallas_call(
        _pool_kernel,
        grid=(N // _ROWS,),
        in_specs=[
            pl.BlockSpec((_ROWS, f), lambda i: (i, 0)),
            pl.BlockSpec((_ROWS, f), lambda i: (i, 0)),
        ],
        out_specs=pl.BlockSpec((G, f), lambda i: (0, 0)),
        out_shape=jax.ShapeDtypeStruct((G, f), jnp.float32),
    )(h, seg)


def kernel(x, edge_index, batch, W_l1, b_l1, W_r1, W_l2, b_l2, W_r2,
           Wq1, bq1, Wk1, bk1, Wv1, bv1, Ws1, bs1,
           Wq2, bq2, Wk2, bk2, Wv2, bv2, Ws2, bs2,
           Wp1, bp1, Wp2, bp2):
    src = edge_index[0]
    dst = edge_index[1]
    h = _sage(x, src, dst, W_l1, b_l1, W_r1)
    h = _sage(h, src, dst, W_l2, b_l2, W_r2)
    h = _tconv(h, src, dst, Wq1, bq1, Wk1, bk1, Wv1, bv1, Ws1, bs1)
    h = _tconv(h, src, dst, Wq2, bq2, Wk2, bk2, Wv2, bv2, Ws2, bs2)
    h = _mm(h, Wp1, bp1, act="relu")
    h = _mm(h, Wp2, bp2, act="sigmoid")
    return _pool(h, batch)
